# baseline (speedup 1.0000x reference)
import jax
import jax.numpy as jnp
from jax import lax
from jax.experimental import pallas as pl
from jax.experimental.pallas import tpu as pltpu
from jax.experimental.pallas import tpu_sc as plsc

EMB = 64
ROW = 64
NUM_CORES = 2
NUM_SUBCORES = 16
NUM_WORKERS = NUM_CORES * NUM_SUBCORES
CHUNK = 128
LANES = 16


def _sc_embedding(table_pad, idx3):
    n_chunks = idx3.shape[1]
    b_per_w = n_chunks * CHUNK
    total = NUM_WORKERS * b_per_w
    mesh = plsc.VectorSubcoreMesh(core_axis_name="c", subcore_axis_name="s")

    def body(table_hbm, idx_hbm, out_hbm, mask_hbm,
             idx_v, mask_v, rows_v,
             sem_g0, sem_g1, sem_g2, sem_g3,
             sem_w0, sem_w1, sem_w2, sem_w3, sem_m):
        sems_g = (sem_g0, sem_g1, sem_g2, sem_g3)
        sems_w = (sem_w0, sem_w1, sem_w2, sem_w3)
        wid = lax.axis_index("s") * NUM_CORES + lax.axis_index("c")
        base = wid * b_per_w

        pltpu.sync_copy(idx_hbm.at[wid], idx_v)

        def start_gather(j, b):
            pltpu.async_copy(table_hbm.at[idx_v.at[j]], rows_v.at[b],
                             sems_g[b])

        def wait_gather(j, b):
            pltpu.make_async_copy(table_hbm.at[idx_v.at[j]], rows_v.at[b],
                                  sems_g[b]).wait()

        def start_write(j, b):
            pltpu.async_copy(rows_v.at[b],
                             out_hbm.at[pl.ds(base + j * CHUNK, CHUNK)],
                             sems_w[b])

        def wait_write(j, b):
            pltpu.make_async_copy(rows_v.at[b],
                                  out_hbm.at[pl.ds(base + j * CHUNK, CHUNK)],
                                  sems_w[b]).wait()

        def compute_mask(j):
            for i in range(CHUNK // LANES):
                v = idx_v[j, pl.ds(i * LANES, LANES)]
                mask_v[j, pl.ds(i * LANES, LANES)] = jnp.minimum(
                    v, 1).astype(jnp.float32)

        start_gather(0, 0)
        start_gather(1, 1)
        for j in (0, 1):
            start_gather(j + 2, j + 2)
            compute_mask(j)
            wait_gather(j, j)
            start_write(j, j)

        def group(g, _):
            for b in range(4):
                j = 2 + g * 4 + b
                jb = (2 + b) % 4
                lb = b
                wait_write(j - 2, lb)
                start_gather(j + 2, lb)
                compute_mask(j)
                wait_gather(j, jb)
                start_write(j, jb)
            return _

        lax.fori_loop(0, (n_chunks - 4) // 4, group, None)

        for j in (n_chunks - 2, n_chunks - 1):
            b = j % 4
            compute_mask(j)
            wait_gather(j, b)
            start_write(j, b)

        mask_copy = pltpu.make_async_copy(mask_v, mask_hbm.at[wid], sem_m)
        mask_copy.start()
        for j in range(n_chunks - 4, n_chunks):
            wait_write(j, j % 4)
        mask_copy.wait()

    run = pl.kernel(
        body,
        out_type=[
            jax.ShapeDtypeStruct((total, ROW), jnp.float32),
            jax.ShapeDtypeStruct((NUM_WORKERS, n_chunks, CHUNK),
                                 jnp.float32),
        ],
        mesh=mesh,
        compiler_params=pltpu.CompilerParams(use_tc_tiling_on_sc=False),
        scratch_types=[
            pltpu.VMEM((n_chunks, CHUNK), jnp.int32),
            pltpu.VMEM((n_chunks, CHUNK), jnp.float32),
            pltpu.VMEM((4, CHUNK, ROW), jnp.float32),
        ] + [pltpu.SemaphoreType.DMA] * 9,
    )
    return run(table_pad, idx3)


def kernel(x, table):
    batch, seq = x.shape
    total = batch * seq
    n_chunks = total // (NUM_WORKERS * CHUNK)
    idx3 = x.astype(jnp.int32).reshape(NUM_WORKERS, n_chunks, CHUNK)
    rows, mask = _sc_embedding(table, idx3)
    emb = rows.reshape(batch, seq, EMB)
    return emb, mask.reshape(batch, seq)


# speedup vs baseline: 1.2992x; 1.2992x over previous
import jax
import jax.numpy as jnp
from jax import lax
from jax.experimental import pallas as pl
from jax.experimental.pallas import tpu as pltpu
from jax.experimental.pallas import tpu_sc as plsc

EMB = 64
ROW = 128
NUM_CORES = 2
NUM_SUBCORES = 16
NUM_WORKERS = NUM_CORES * NUM_SUBCORES
CHUNK = 128
LANES = 16


def _sc_embedding(table_pad, idx3):
    n_chunks = idx3.shape[1]
    b_per_w = n_chunks * CHUNK
    total = NUM_WORKERS * b_per_w
    mesh = plsc.VectorSubcoreMesh(core_axis_name="c", subcore_axis_name="s")

    def body(table_hbm, idx_hbm, out_hbm, mask_hbm,
             idx_v, mask_v, rows_v,
             sem_g0, sem_g1, sem_g2, sem_g3,
             sem_w0, sem_w1, sem_w2, sem_w3, sem_m):
        sems_g = (sem_g0, sem_g1, sem_g2, sem_g3)
        sems_w = (sem_w0, sem_w1, sem_w2, sem_w3)
        wid = lax.axis_index("s") * NUM_CORES + lax.axis_index("c")
        base = wid * b_per_w

        pltpu.sync_copy(idx_hbm.at[wid], idx_v)

        def start_gather(j, b):
            pltpu.async_copy(table_hbm.at[idx_v.at[j]], rows_v.at[b],
                             sems_g[b])

        def wait_gather(j, b):
            pltpu.make_async_copy(table_hbm.at[idx_v.at[j]], rows_v.at[b],
                                  sems_g[b]).wait()

        def start_write(j, b):
            pltpu.async_copy(rows_v.at[b],
                             out_hbm.at[pl.ds(base + j * CHUNK, CHUNK)],
                             sems_w[b])

        def wait_write(j, b):
            pltpu.make_async_copy(rows_v.at[b],
                                  out_hbm.at[pl.ds(base + j * CHUNK, CHUNK)],
                                  sems_w[b]).wait()

        def compute_mask(j):
            for i in range(CHUNK // LANES):
                v = idx_v[j, pl.ds(i * LANES, LANES)]
                mask_v[j, pl.ds(i * LANES, LANES)] = jnp.minimum(
                    v, 1).astype(jnp.float32)

        start_gather(0, 0)
        start_gather(1, 1)
        for j in (0, 1):
            start_gather(j + 2, j + 2)
            compute_mask(j)
            wait_gather(j, j)
            start_write(j, j)

        def group(g, _):
            for b in range(4):
                j = 2 + g * 4 + b
                jb = (2 + b) % 4
                lb = b
                wait_write(j - 2, lb)
                start_gather(j + 2, lb)
                compute_mask(j)
                wait_gather(j, jb)
                start_write(j, jb)
            return _

        lax.fori_loop(0, (n_chunks - 4) // 4, group, None)

        for j in (n_chunks - 2, n_chunks - 1):
            b = j % 4
            compute_mask(j)
            wait_gather(j, b)
            start_write(j, b)

        mask_copy = pltpu.make_async_copy(mask_v, mask_hbm.at[wid], sem_m)
        mask_copy.start()
        for j in range(n_chunks - 4, n_chunks):
            wait_write(j, j % 4)
        mask_copy.wait()

    run = pl.kernel(
        body,
        out_type=[
            jax.ShapeDtypeStruct((total, ROW), jnp.float32),
            jax.ShapeDtypeStruct((NUM_WORKERS, n_chunks, CHUNK),
                                 jnp.float32),
        ],
        mesh=mesh,
        compiler_params=pltpu.CompilerParams(use_tc_tiling_on_sc=False),
        scratch_types=[
            pltpu.VMEM((n_chunks, CHUNK), jnp.int32),
            pltpu.VMEM((n_chunks, CHUNK), jnp.float32),
            pltpu.VMEM((4, CHUNK, ROW), jnp.float32),
        ] + [pltpu.SemaphoreType.DMA] * 9,
    )
    return run(table_pad, idx3)


def _pad_table_tc(table):
    V = table.shape[0]
    B = 2048
    grid = (V + B - 1) // B

    def body(tt_ref, out_ref):
        t = tt_ref[...].T
        out_ref[...] = jnp.concatenate(
            [t, jnp.zeros((B, ROW - EMB), jnp.float32)], axis=1)

    return pl.pallas_call(
        body,
        grid=(grid,),
        in_specs=[pl.BlockSpec((EMB, B), lambda i: (0, i))],
        out_specs=pl.BlockSpec((B, ROW), lambda i: (i, 0)),
        out_shape=jax.ShapeDtypeStruct((V, ROW), jnp.float32),
    )(table.T)


def kernel(x, table):
    batch, seq = x.shape
    total = batch * seq
    n_chunks = total // (NUM_WORKERS * CHUNK)
    idx3 = x.astype(jnp.int32).reshape(NUM_WORKERS, n_chunks, CHUNK)
    table_pad = _pad_table_tc(table)
    rows, mask = _sc_embedding(table_pad, idx3)
    emb = rows[:, :EMB].reshape(batch, seq, EMB)
    return emb, mask.reshape(batch, seq)


# speedup vs baseline: 1.4086x; 1.0842x over previous
import jax
import jax.numpy as jnp
from jax import lax
from jax.experimental import pallas as pl
from jax.experimental.pallas import tpu as pltpu
from jax.experimental.pallas import tpu_sc as plsc

EMB = 64
ROW = 128
NUM_CORES = 2
NUM_SUBCORES = 16
NUM_WORKERS = NUM_CORES * NUM_SUBCORES
CHUNK = 128
LANES = 16


def _sc_embedding(table_pad, idx3):
    n_chunks = idx3.shape[1]
    b_per_w = n_chunks * CHUNK
    total = NUM_WORKERS * b_per_w
    mesh = plsc.VectorSubcoreMesh(core_axis_name="c", subcore_axis_name="s")

    def body(table_hbm, idx_hbm, out_hbm, mask_hbm,
             idx_v, mask_v, rows_v,
             sem_g0, sem_g1, sem_g2, sem_g3,
             sem_w0, sem_w1, sem_w2, sem_w3, sem_m):
        sems_g = (sem_g0, sem_g1, sem_g2, sem_g3)
        sems_w = (sem_w0, sem_w1, sem_w2, sem_w3)
        wid = lax.axis_index("s") * NUM_CORES + lax.axis_index("c")
        base = wid * b_per_w

        pltpu.sync_copy(idx_hbm.at[wid], idx_v)

        def start_gather(j, b):
            pltpu.async_copy(table_hbm.at[idx_v.at[j]], rows_v.at[b],
                             sems_g[b])

        def wait_gather(j, b):
            pltpu.make_async_copy(table_hbm.at[idx_v.at[j]], rows_v.at[b],
                                  sems_g[b]).wait()

        def start_write(j, b):
            pltpu.async_copy(rows_v.at[b, :, pl.ds(0, EMB)],
                             out_hbm.at[pl.ds(base + j * CHUNK, CHUNK),
                                        pl.ds(0, EMB)],
                             sems_w[b])

        def wait_write(j, b):
            pltpu.make_async_copy(rows_v.at[b, :, pl.ds(0, EMB)],
                                  out_hbm.at[pl.ds(base + j * CHUNK, CHUNK),
                                             pl.ds(0, EMB)],
                                  sems_w[b]).wait()

        def compute_mask(j):
            for i in range(CHUNK // LANES):
                v = idx_v[j, pl.ds(i * LANES, LANES)]
                mask_v[j, pl.ds(i * LANES, LANES)] = jnp.minimum(
                    v, 1).astype(jnp.float32)

        start_gather(0, 0)
        start_gather(1, 1)
        for j in (0, 1):
            start_gather(j + 2, j + 2)
            compute_mask(j)
            wait_gather(j, j)
            start_write(j, j)

        def group(g, _):
            for b in range(4):
                j = 2 + g * 4 + b
                jb = (2 + b) % 4
                lb = b
                wait_write(j - 2, lb)
                start_gather(j + 2, lb)
                compute_mask(j)
                wait_gather(j, jb)
                start_write(j, jb)
            return _

        lax.fori_loop(0, (n_chunks - 4) // 4, group, None)

        for j in (n_chunks - 2, n_chunks - 1):
            b = j % 4
            compute_mask(j)
            wait_gather(j, b)
            start_write(j, b)

        mask_copy = pltpu.make_async_copy(mask_v, mask_hbm.at[wid], sem_m)
        mask_copy.start()
        for j in range(n_chunks - 4, n_chunks):
            wait_write(j, j % 4)
        mask_copy.wait()

    run = pl.kernel(
        body,
        out_type=[
            jax.ShapeDtypeStruct((total, ROW), jnp.float32),
            jax.ShapeDtypeStruct((NUM_WORKERS, n_chunks, CHUNK),
                                 jnp.float32),
        ],
        mesh=mesh,
        compiler_params=pltpu.CompilerParams(use_tc_tiling_on_sc=False),
        scratch_types=[
            pltpu.VMEM((n_chunks, CHUNK), jnp.int32),
            pltpu.VMEM((n_chunks, CHUNK), jnp.float32),
            pltpu.VMEM((4, CHUNK, ROW), jnp.float32),
        ] + [pltpu.SemaphoreType.DMA] * 9,
    )
    return run(table_pad, idx3)


def _pad_table_tc(table):
    V = table.shape[0]
    B = 2048
    grid = (V + B - 1) // B

    def body(tt_ref, out_ref):
        t = tt_ref[...].T
        out_ref[...] = jnp.concatenate(
            [t, jnp.zeros((B, ROW - EMB), jnp.float32)], axis=1)

    return pl.pallas_call(
        body,
        grid=(grid,),
        in_specs=[pl.BlockSpec((EMB, B), lambda i: (0, i))],
        out_specs=pl.BlockSpec((B, ROW), lambda i: (i, 0)),
        out_shape=jax.ShapeDtypeStruct((V, ROW), jnp.float32),
    )(table.T)


def kernel(x, table):
    batch, seq = x.shape
    total = batch * seq
    n_chunks = total // (NUM_WORKERS * CHUNK)
    idx3 = x.astype(jnp.int32).reshape(NUM_WORKERS, n_chunks, CHUNK)
    table_pad = _pad_table_tc(table)
    rows, mask = _sc_embedding(table_pad, idx3)
    emb = rows[:, :EMB].reshape(batch, seq, EMB)
    return emb, mask.reshape(batch, seq)
